# baseline (speedup 1.0000x reference)
import functools

import jax
import jax.numpy as jnp
from jax import lax
from jax.experimental import pallas as pl
from jax.experimental.pallas import tpu as pltpu
from jax.experimental.pallas import tpu_sc as plsc

_B = 64
_C = 4
_H = 224
_W = 224
_BB = 16
_NPIX = _B * _H * _W

_NC = 2
_NS = 16
_NW = _NC * _NS
_BPW = _B // _NW
_RCH = 32
_NCHUNK = _H // _RCH
_NREG = _RCH * _W // 16


def _focal_kernel(x_ref, y_ref, out_ref, acc_ref):
    step = pl.program_id(0)

    @pl.when(step == 0)
    def _init():
        acc_ref[0] = 0.0

    p = x_ref[:, 0]
    t = y_ref[:, 0]
    logp = jnp.maximum(jnp.log(p), -100.0)
    log1mp = jnp.maximum(jnp.log(1.0 - p), -100.0)
    bce = -(t * logp + (1.0 - t) * log1mp)
    pt = jnp.exp(-bce)
    one_m_pt = 1.0 - pt
    f = 0.9 * one_m_pt * one_m_pt * bce
    acc_ref[0] += jnp.sum(f)

    @pl.when(step == pl.num_programs(0) - 1)
    def _fini():
        out_ref[0] = acc_ref[0]


def _tc_focal(inputs, target):
    spec = pl.BlockSpec((_BB, 1, _H, _W), lambda b: (b, 0, 0, 0))
    out = pl.pallas_call(
        _focal_kernel,
        grid=(_B // _BB,),
        in_specs=[spec, spec],
        out_specs=pl.BlockSpec(memory_space=pltpu.SMEM),
        out_shape=jax.ShapeDtypeStruct((1,), jnp.float32),
        scratch_shapes=[pltpu.SMEM((1,), jnp.float32)],
        compiler_params=pltpu.CompilerParams(has_side_effects=False),
    )(inputs, target)
    return out[0]


_sc_mesh = plsc.VectorSubcoreMesh(core_axis_name="c", subcore_axis_name="s")


@functools.partial(
    pl.kernel,
    mesh=_sc_mesh,
    out_type=jax.ShapeDtypeStruct((2 * _NW, 16), jnp.float32),
    scratch_types=(
        [pltpu.VMEM((_RCH, _W), jnp.float32) for _ in range(14)]
        + [pltpu.SemaphoreType.DMA, pltpu.SemaphoreType.DMA]
    ),
)
def _sc_mse(x_hbm, y_hbm, out_hbm, *scr):
    bufs = [scr[:7], scr[7:14]]
    sems = scr[14:16]
    w = lax.axis_index("s") * _NC + lax.axis_index("c")

    chunks = [(i, ck) for i in range(_BPW) for ck in range(_NCHUNK)]

    def issue(idx, bset, sem):
        i, ck = chunks[idx]
        b = w * _BPW + i
        rows = pl.ds(ck * _RCH, _RCH)
        hs = []
        hs.append(pltpu.async_copy(y_hbm.at[b, 0, rows], bset[0], sem))
        for ch in range(1, 4):
            hs.append(pltpu.async_copy(y_hbm.at[b, ch, rows], bset[ch], sem))
            hs.append(pltpu.async_copy(x_hbm.at[b, ch, rows], bset[3 + ch], sem))
        return hs

    msq = jnp.zeros((16,), jnp.float32)
    cnt = jnp.zeros((16,), jnp.float32)
    pend = issue(0, bufs[0], sems[0])
    for idx in range(len(chunks)):
        cur = bufs[idx % 2]
        for h in pend:
            h.wait()
        if idx + 1 < len(chunks):
            pend = issue(idx + 1, bufs[(idx + 1) % 2], sems[(idx + 1) % 2])
        else:
            pend = []

        mbuf = cur[0]

        def row_body(r, carry):
            a_msq, a_cnt = carry
            for j in range(_W // 16):
                cols = pl.ds(j * 16, 16)
                m = mbuf[r, cols] != 0.0
                a_cnt = a_cnt + jnp.where(m, 1.0, 0.0)
                for ch in range(1, 4):
                    d = cur[ch][r, cols] - cur[3 + ch][r, cols]
                    a_msq = a_msq + jnp.where(m, d * d, 0.0)
            return (a_msq, a_cnt)

        msq, cnt = lax.fori_loop(0, _RCH, row_body, (msq, cnt))

    scr[0][0, pl.ds(0, 16)] = msq
    scr[0][1, pl.ds(0, 16)] = cnt
    pltpu.sync_copy(scr[0].at[0, pl.ds(0, 16)], out_hbm.at[w])
    pltpu.sync_copy(scr[0].at[1, pl.ds(0, 16)], out_hbm.at[_NW + w])


def kernel(inputs, target):
    parts = _sc_mse(inputs, target)
    fsum = _tc_focal(inputs, target)
    msq = jnp.sum(parts[:_NW])
    cnt = jnp.sum(parts[_NW:])
    return fsum / _NPIX + msq / (cnt * 3.0)


# speedup vs baseline: 1.8627x; 1.8627x over previous
import jax
import jax.numpy as jnp
from jax.experimental import pallas as pl
from jax.experimental.pallas import tpu as pltpu

_B = 64
_C = 4
_H = 224
_W = 224
_BB = 8
_NPIX = _B * _H * _W


def _loss_kernel(x_ref, y_ref, out_ref, acc_ref):
    step = pl.program_id(0)

    @pl.when(step == 0)
    def _init():
        acc_ref[0] = 0.0
        acc_ref[1] = 0.0
        acc_ref[2] = 0.0

    p = x_ref[:, 0]
    t = y_ref[:, 0]
    logp = jnp.maximum(jnp.log(p), -100.0)
    log1mp = jnp.maximum(jnp.log(1.0 - p), -100.0)
    bce = -(t * logp + (1.0 - t) * log1mp)
    pt = jnp.exp(-bce)
    one_m_pt = 1.0 - pt
    f = 0.9 * one_m_pt * one_m_pt * bce

    mask = t != 0.0
    cnt = jnp.sum(mask.astype(jnp.float32))

    d = y_ref[:, 1:] - x_ref[:, 1:]
    sq = d * d
    msq = jnp.sum(jnp.where(mask[:, None], sq, 0.0))

    acc_ref[0] += jnp.sum(f)
    acc_ref[1] += msq
    acc_ref[2] += cnt

    @pl.when(step == pl.num_programs(0) - 1)
    def _fini():
        out_ref[0] = acc_ref[0] / _NPIX + acc_ref[1] / (acc_ref[2] * 3.0)


def kernel(inputs, target):
    spec = pl.BlockSpec((_BB, _C, _H, _W), lambda b: (b, 0, 0, 0))
    out = pl.pallas_call(
        _loss_kernel,
        grid=(_B // _BB,),
        in_specs=[spec, spec],
        out_specs=pl.BlockSpec(memory_space=pltpu.SMEM),
        out_shape=jax.ShapeDtypeStruct((1,), jnp.float32),
        scratch_shapes=[pltpu.SMEM((3,), jnp.float32)],
    )(inputs, target)
    return out[0]


# speedup vs baseline: 2.0396x; 1.0950x over previous
import jax
import jax.numpy as jnp
from jax.experimental import pallas as pl
from jax.experimental.pallas import tpu as pltpu

_B = 64
_C = 4
_H = 224
_W = 224
_BB = 8
_NPIX = _B * _H * _W


def _loss_kernel(x_ref, y_ref, out_ref, acc_ref):
    step = pl.program_id(0)

    @pl.when(step == 0)
    def _init():
        acc_ref[0] = 0.0
        acc_ref[1] = 0.0
        acc_ref[2] = 0.0

    p = x_ref[:, 0]
    t = y_ref[:, 0]
    logp = jnp.maximum(jnp.log(p), -100.0)
    log1mp = jnp.maximum(jnp.log(1.0 - p), -100.0)
    nb = log1mp + t * (logp - log1mp)
    pt = jnp.exp(nb)
    one_m_pt = 1.0 - pt
    g = one_m_pt * one_m_pt * nb

    m = jnp.where(t != 0.0, 1.0, 0.0)

    d1 = y_ref[:, 1] - x_ref[:, 1]
    d2 = y_ref[:, 2] - x_ref[:, 2]
    d3 = y_ref[:, 3] - x_ref[:, 3]
    s = d1 * d1 + d2 * d2 + d3 * d3

    acc_ref[0] += jnp.sum(g)
    acc_ref[1] += jnp.sum(m * s)
    acc_ref[2] += jnp.sum(m)

    @pl.when(step == pl.num_programs(0) - 1)
    def _fini():
        out_ref[0] = -0.9 * acc_ref[0] / _NPIX + acc_ref[1] / (acc_ref[2] * 3.0)


def kernel(inputs, target):
    spec = pl.BlockSpec((_BB, _C, _H, _W), lambda b: (b, 0, 0, 0))
    out = pl.pallas_call(
        _loss_kernel,
        grid=(_B // _BB,),
        in_specs=[spec, spec],
        out_specs=pl.BlockSpec(memory_space=pltpu.SMEM),
        out_shape=jax.ShapeDtypeStruct((1,), jnp.float32),
        scratch_shapes=[pltpu.SMEM((3,), jnp.float32)],
    )(inputs, target)
    return out[0]
